# Initial kernel scaffold; baseline (speedup 1.0000x reference)
import jax
import jax.numpy as jnp
from jax.experimental import pallas as pl


def kernel(x, edge_index, W_self, b_self, W_neigh, b_neigh):
    raise NotImplementedError("write your pallas kernel here")


# speedup vs baseline: 2.6789x; 2.6789x over previous
import functools
import jax
import jax.numpy as jnp
from jax import lax
from jax.experimental import pallas as pl
from jax.experimental.pallas import tpu as pltpu
from jax.experimental.pallas import tpu_sc as plsc

N = 10000
D = 256
Q = 64
E = 160000
EPS = 1e-7

NS = 16
CHUNK = 128
NCHUNK = 80
NSEC = 4
SCHUNK = NCHUNK // NSEC
E_PAD = NS * NCHUNK * CHUNK
TRASH = N
NPAD = 10112
RPW = NPAD // NS

ROW_BLK = 1000


def _logmap_body(x_ref, q0_ref, q1_ref, q2_ref, q3_ref):
    x = x_ref[...]
    nrm = jnp.sqrt(jnp.sum(x * x, axis=1, keepdims=True))
    nrm = jnp.maximum(nrm, EPS)
    y = jnp.minimum(nrm, 1.0 - 1e-5)
    scale = 0.5 * jnp.log((1.0 + y) / (1.0 - y)) / nrm
    xt = x * scale
    q0_ref[...] = xt[:, 0 * Q:1 * Q]
    q1_ref[...] = xt[:, 1 * Q:2 * Q]
    q2_ref[...] = xt[:, 2 * Q:3 * Q]
    q3_ref[...] = xt[:, 3 * Q:4 * Q]


_Q_SPEC = pl.BlockSpec((ROW_BLK, Q), lambda i: (i, 0))
_Q_SHAPE = jax.ShapeDtypeStruct((N, Q), jnp.float32)

_logmap = pl.pallas_call(
    _logmap_body,
    grid=(N // ROW_BLK,),
    in_specs=[pl.BlockSpec((ROW_BLK, D), lambda i: (i, 0))],
    out_specs=[_Q_SPEC, _Q_SPEC, _Q_SPEC, _Q_SPEC],
    out_shape=[_Q_SHAPE, _Q_SHAPE, _Q_SHAPE, _Q_SHAPE],
)


@functools.partial(
    pl.kernel,
    mesh=plsc.VectorSubcoreMesh(core_axis_name="c", subcore_axis_name="s"),
    compiler_params=pltpu.CompilerParams(use_tc_tiling_on_sc=False),
    out_type=[
        jax.ShapeDtypeStruct((NPAD, Q), jnp.float32),
        jax.ShapeDtypeStruct((NPAD, Q), jnp.float32),
        jax.ShapeDtypeStruct((NPAD, Q), jnp.float32),
        jax.ShapeDtypeStruct((NPAD, Q), jnp.float32),
        jax.ShapeDtypeStruct((NPAD, 16), jnp.float32),
    ],
    scratch_types=[
        pltpu.VMEM((SCHUNK, CHUNK), jnp.int32),
        pltpu.VMEM((SCHUNK, CHUNK), jnp.int32),
        pltpu.VMEM((CHUNK, Q), jnp.float32),
        pltpu.VMEM((CHUNK, 16), jnp.float32),
        pltpu.VMEM_SHARED((NPAD, Q), jnp.float32),
        pltpu.VMEM_SHARED((NPAD, 16), jnp.float32),
        pltpu.SemaphoreType.DMA,
    ],
)
def _sc_agg(xq0_hbm, xq1_hbm, xq2_hbm, xq3_hbm, src_hbm, dst_hbm,
            zrow_hbm, zdeg_hbm, ones_hbm,
            agg0_hbm, agg1_hbm, agg2_hbm, agg3_hbm, deg_hbm,
            src_v, dst_v, rows_v, ones_v, acc_sh, deg_sh, sem):
    c = lax.axis_index("c")
    s = lax.axis_index("s")
    rows = pl.ds(s * RPW, RPW)

    pltpu.sync_copy(ones_hbm, ones_v)

    def one_pass(xq_hbm, aggq_hbm, do_deg):
        pltpu.sync_copy(zrow_hbm, acc_sh.at[rows])
        if do_deg:
            pltpu.sync_copy(zdeg_hbm, deg_sh.at[rows])
        plsc.subcore_barrier()

        for sec in range(NSEC):
            pltpu.sync_copy(src_hbm.at[s * NSEC + sec], src_v)
            pltpu.sync_copy(dst_hbm.at[s * NSEC + sec], dst_v)

            def body(j, carry):
                pltpu.async_copy(xq_hbm.at[src_v.at[j]], rows_v, sem).wait()
                pltpu.sync_copy(rows_v, acc_sh.at[dst_v.at[j]], add=True)
                if do_deg:
                    pltpu.sync_copy(ones_v, deg_sh.at[dst_v.at[j]], add=True)
                return carry
            lax.fori_loop(0, SCHUNK, body, 0)

        plsc.subcore_barrier()
        pltpu.sync_copy(acc_sh.at[rows], aggq_hbm.at[rows])
        if do_deg:
            pltpu.sync_copy(deg_sh.at[rows], deg_hbm.at[rows])

    @pl.when(c == 0)
    def _():
        one_pass(xq0_hbm, agg0_hbm, False)
        one_pass(xq1_hbm, agg1_hbm, False)

    @pl.when(c == 1)
    def _():
        one_pass(xq2_hbm, agg2_hbm, True)
        one_pass(xq3_hbm, agg3_hbm, False)


def _combine_body(q0_ref, q1_ref, q2_ref, q3_ref,
                  a0_ref, a1_ref, a2_ref, a3_ref, deg_ref,
                  ws_ref, wn_ref, b_ref, o_ref):
    inv = 1.0 / jnp.maximum(deg_ref[:, 0:1], 1.0)
    xt = jnp.concatenate(
        [q0_ref[...], q1_ref[...], q2_ref[...], q3_ref[...]], axis=1)
    ag = jnp.concatenate(
        [a0_ref[...], a1_ref[...], a2_ref[...], a3_ref[...]], axis=1)
    h = (jnp.dot(xt, ws_ref[...], preferred_element_type=jnp.float32)
         + jnp.dot(ag * inv, wn_ref[...], preferred_element_type=jnp.float32)
         + b_ref[...])
    nrm = jnp.sqrt(jnp.sum(h * h, axis=1, keepdims=True))
    nrm = jnp.maximum(nrm, EPS)
    o_ref[...] = jnp.tanh(nrm) * h / nrm


_W_SPEC = pl.BlockSpec((D, D), lambda i: (0, 0))

_combine = pl.pallas_call(
    _combine_body,
    grid=(N // ROW_BLK,),
    in_specs=[
        _Q_SPEC, _Q_SPEC, _Q_SPEC, _Q_SPEC,
        _Q_SPEC, _Q_SPEC, _Q_SPEC, _Q_SPEC,
        pl.BlockSpec((ROW_BLK, 16), lambda i: (i, 0)),
        _W_SPEC, _W_SPEC,
        pl.BlockSpec((1, D), lambda i: (0, 0)),
    ],
    out_specs=pl.BlockSpec((ROW_BLK, D), lambda i: (i, 0)),
    out_shape=jax.ShapeDtypeStruct((N, D), jnp.float32),
)


def kernel(x, edge_index, W_self, b_self, W_neigh, b_neigh):
    src = edge_index[0].astype(jnp.int32)
    dst = edge_index[1].astype(jnp.int32)
    pad = E_PAD - E
    src2 = jnp.concatenate([src, jnp.zeros((pad,), jnp.int32)]).reshape(
        NS * NSEC, SCHUNK, CHUNK)
    dst2 = jnp.concatenate([dst, jnp.full((pad,), TRASH, jnp.int32)]).reshape(
        NS * NSEC, SCHUNK, CHUNK)

    q0, q1, q2, q3 = _logmap(x)

    zrow = jnp.zeros((RPW, Q), jnp.float32)
    zdeg = jnp.zeros((RPW, 16), jnp.float32)
    ones = jnp.ones((CHUNK, 16), jnp.float32)
    a0, a1, a2, a3, deg = _sc_agg(q0, q1, q2, q3, src2, dst2, zrow, zdeg, ones)

    bias = (b_self + b_neigh).reshape(1, D)
    return _combine(q0, q1, q2, q3, a0, a1, a2, a3, deg,
                    W_self.T, W_neigh.T, bias)


# speedup vs baseline: 3.3702x; 1.2581x over previous
import functools
import jax
import jax.numpy as jnp
from jax import lax
from jax.experimental import pallas as pl
from jax.experimental.pallas import tpu as pltpu
from jax.experimental.pallas import tpu_sc as plsc

N = 10000
D = 256
Q = 64
E = 160000
EPS = 1e-7

NS = 16
CHUNK = 128
NCHUNK = 80
NBUF = 4
NROUND = NCHUNK // NBUF
E_PAD = NS * NCHUNK * CHUNK
TRASH = N
NPAD = 10112
RPW = NPAD // NS

ROW_BLK = 1000


def _logmap_body(x_ref, q0_ref, q1_ref, q2_ref, q3_ref):
    x = x_ref[...]
    nrm = jnp.sqrt(jnp.sum(x * x, axis=1, keepdims=True))
    nrm = jnp.maximum(nrm, EPS)
    y = jnp.minimum(nrm, 1.0 - 1e-5)
    scale = 0.5 * jnp.log((1.0 + y) / (1.0 - y)) / nrm
    xt = x * scale
    q0_ref[...] = xt[:, 0 * Q:1 * Q]
    q1_ref[...] = xt[:, 1 * Q:2 * Q]
    q2_ref[...] = xt[:, 2 * Q:3 * Q]
    q3_ref[...] = xt[:, 3 * Q:4 * Q]


_Q_SPEC = pl.BlockSpec((ROW_BLK, Q), lambda i: (i, 0))
_Q_SHAPE = jax.ShapeDtypeStruct((N, Q), jnp.float32)

_logmap = pl.pallas_call(
    _logmap_body,
    grid=(N // ROW_BLK,),
    in_specs=[pl.BlockSpec((ROW_BLK, D), lambda i: (i, 0))],
    out_specs=[_Q_SPEC, _Q_SPEC, _Q_SPEC, _Q_SPEC],
    out_shape=[_Q_SHAPE, _Q_SHAPE, _Q_SHAPE, _Q_SHAPE],
)


@functools.partial(
    pl.kernel,
    mesh=plsc.VectorSubcoreMesh(core_axis_name="c", subcore_axis_name="s"),
    compiler_params=pltpu.CompilerParams(use_tc_tiling_on_sc=False),
    out_type=[
        jax.ShapeDtypeStruct((NPAD, Q), jnp.float32),
        jax.ShapeDtypeStruct((NPAD, Q), jnp.float32),
        jax.ShapeDtypeStruct((NPAD, Q), jnp.float32),
        jax.ShapeDtypeStruct((NPAD, Q), jnp.float32),
        jax.ShapeDtypeStruct((NPAD, 16), jnp.float32),
        jax.ShapeDtypeStruct((NPAD, 16), jnp.float32),
    ],
    scratch_types=[
        pltpu.VMEM((NCHUNK, CHUNK), jnp.int32),
        pltpu.VMEM((NCHUNK, CHUNK), jnp.int32),
        pltpu.VMEM((CHUNK, Q), jnp.float32),
        pltpu.VMEM((CHUNK, Q), jnp.float32),
        pltpu.VMEM((CHUNK, Q), jnp.float32),
        pltpu.VMEM((CHUNK, Q), jnp.float32),
        pltpu.VMEM((CHUNK, 16), jnp.float32),
        pltpu.VMEM_SHARED((NPAD, Q), jnp.float32),
        pltpu.VMEM_SHARED((NPAD, 16), jnp.float32),
        pltpu.SemaphoreType.DMA,
        pltpu.SemaphoreType.DMA,
        pltpu.SemaphoreType.DMA,
        pltpu.SemaphoreType.DMA,
        pltpu.SemaphoreType.DMA,
        pltpu.SemaphoreType.DMA,
        pltpu.SemaphoreType.DMA,
        pltpu.SemaphoreType.DMA,
        pltpu.SemaphoreType.DMA,
    ],
)
def _sc_agg(xq0_hbm, xq1_hbm, xq2_hbm, xq3_hbm, src_hbm, dst_hbm,
            zrow_hbm, zdeg_hbm, ones_hbm,
            agg0_hbm, agg1_hbm, agg2_hbm, agg3_hbm, dega_hbm, degb_hbm,
            src_v, dst_v, b0, b1, b2, b3, ones_v, acc_sh, deg_sh,
            g0, g1, g2, g3, s0, s1, s2, s3, dsem):
    c = lax.axis_index("c")
    s = lax.axis_index("s")
    rows = pl.ds(s * RPW, RPW)
    bufs = [b0, b1, b2, b3]
    gsems = [g0, g1, g2, g3]
    ssems = [s0, s1, s2, s3]

    pltpu.sync_copy(ones_hbm, ones_v)

    def one_pass(xq_hbm, agg_hbm, deg_rounds, deg_hbm):
        pltpu.sync_copy(src_hbm.at[s], src_v)
        pltpu.sync_copy(dst_hbm.at[s], dst_v)
        pltpu.sync_copy(zrow_hbm, acc_sh.at[rows])
        if deg_rounds is not None:
            pltpu.sync_copy(zdeg_hbm, deg_sh.at[rows])
        plsc.subcore_barrier()

        for b in range(NBUF):
            pltpu.async_copy(xq_hbm.at[src_v.at[b]], bufs[b], gsems[b])

        def rnd(j, carry):
            base = j * NBUF
            for b in range(NBUF):
                pltpu.make_async_copy(
                    xq_hbm.at[pl.ds(0, CHUNK)], bufs[b], gsems[b]).wait()
                pltpu.async_copy(
                    bufs[b], acc_sh.at[dst_v.at[base + b]], ssems[b], add=True)
            if deg_rounds is not None:
                lo, hi = deg_rounds

                @pl.when(jnp.logical_and(j >= lo, j < hi))
                def _():
                    for b in range(NBUF):
                        pltpu.async_copy(
                            ones_v, deg_sh.at[dst_v.at[base + b]], dsem,
                            add=True)
                    for _b in range(NBUF):
                        pltpu.make_async_copy(ones_hbm, ones_v, dsem).wait()
            for b in range(NBUF):
                pltpu.make_async_copy(
                    xq_hbm.at[pl.ds(0, CHUNK)], bufs[b], ssems[b]).wait()

                @pl.when(j < NROUND - 1)
                def _(b=b, base=base):
                    pltpu.async_copy(
                        xq_hbm.at[src_v.at[base + NBUF + b]], bufs[b],
                        gsems[b])
            return carry

        lax.fori_loop(0, NROUND, rnd, 0)
        plsc.subcore_barrier()
        pltpu.sync_copy(acc_sh.at[rows], agg_hbm.at[rows])
        if deg_rounds is not None:
            pltpu.sync_copy(deg_sh.at[rows], deg_hbm.at[rows])

    @pl.when(c == 0)
    def _():
        one_pass(xq0_hbm, agg0_hbm, (0, NROUND // 2), dega_hbm)
        one_pass(xq1_hbm, agg1_hbm, None, None)

    @pl.when(c == 1)
    def _():
        one_pass(xq2_hbm, agg2_hbm, (NROUND // 2, NROUND), degb_hbm)
        one_pass(xq3_hbm, agg3_hbm, None, None)


def _combine_body(q0_ref, q1_ref, q2_ref, q3_ref,
                  a0_ref, a1_ref, a2_ref, a3_ref, dega_ref, degb_ref,
                  ws_ref, wn_ref, b_ref, o_ref):
    deg = dega_ref[:, 0:1] + degb_ref[:, 0:1]
    inv = 1.0 / jnp.maximum(deg, 1.0)
    xt = jnp.concatenate(
        [q0_ref[...], q1_ref[...], q2_ref[...], q3_ref[...]], axis=1)
    ag = jnp.concatenate(
        [a0_ref[...], a1_ref[...], a2_ref[...], a3_ref[...]], axis=1)
    h = (jnp.dot(xt, ws_ref[...], preferred_element_type=jnp.float32)
         + jnp.dot(ag * inv, wn_ref[...], preferred_element_type=jnp.float32)
         + b_ref[...])
    nrm = jnp.sqrt(jnp.sum(h * h, axis=1, keepdims=True))
    nrm = jnp.maximum(nrm, EPS)
    o_ref[...] = jnp.tanh(nrm) * h / nrm


_W_SPEC = pl.BlockSpec((D, D), lambda i: (0, 0))
_DEG_SPEC = pl.BlockSpec((ROW_BLK, 16), lambda i: (i, 0))

_combine = pl.pallas_call(
    _combine_body,
    grid=(N // ROW_BLK,),
    in_specs=[
        _Q_SPEC, _Q_SPEC, _Q_SPEC, _Q_SPEC,
        _Q_SPEC, _Q_SPEC, _Q_SPEC, _Q_SPEC,
        _DEG_SPEC, _DEG_SPEC,
        _W_SPEC, _W_SPEC,
        pl.BlockSpec((1, D), lambda i: (0, 0)),
    ],
    out_specs=pl.BlockSpec((ROW_BLK, D), lambda i: (i, 0)),
    out_shape=jax.ShapeDtypeStruct((N, D), jnp.float32),
)


def kernel(x, edge_index, W_self, b_self, W_neigh, b_neigh):
    src = edge_index[0].astype(jnp.int32)
    dst = edge_index[1].astype(jnp.int32)
    pad = E_PAD - E
    src2 = jnp.concatenate([src, jnp.zeros((pad,), jnp.int32)]).reshape(
        NS, NCHUNK, CHUNK)
    dst2 = jnp.concatenate([dst, jnp.full((pad,), TRASH, jnp.int32)]).reshape(
        NS, NCHUNK, CHUNK)

    q0, q1, q2, q3 = _logmap(x)

    zrow = jnp.zeros((RPW, Q), jnp.float32)
    zdeg = jnp.zeros((RPW, 16), jnp.float32)
    ones = jnp.ones((CHUNK, 16), jnp.float32)
    a0, a1, a2, a3, dega, degb = _sc_agg(
        q0, q1, q2, q3, src2, dst2, zrow, zdeg, ones)

    bias = (b_self + b_neigh).reshape(1, D)
    return _combine(q0, q1, q2, q3, a0, a1, a2, a3, dega, degb,
                    W_self.T, W_neigh.T, bias)


# speedup vs baseline: 3.4502x; 1.0237x over previous
import functools
import jax
import jax.numpy as jnp
from jax import lax
from jax.experimental import pallas as pl
from jax.experimental.pallas import tpu as pltpu
from jax.experimental.pallas import tpu_sc as plsc

N = 10000
D = 256
Q = 64
E = 160000
EPS = 1e-7

NS = 16
CHUNK = 64
NCHUNK = 160
NBUF = 4
NROUND = NCHUNK // (2 * NBUF)
E_PAD = NS * NCHUNK * CHUNK
TRASH = N
NPAD = 10112
RPW = NPAD // NS

ROW_BLK = 1000


def _logmap_body(x_ref, q0_ref, q1_ref, q2_ref, q3_ref):
    x = x_ref[...]
    nrm = jnp.sqrt(jnp.sum(x * x, axis=1, keepdims=True))
    nrm = jnp.maximum(nrm, EPS)
    y = jnp.minimum(nrm, 1.0 - 1e-5)
    scale = 0.5 * jnp.log((1.0 + y) / (1.0 - y)) / nrm
    xt = x * scale
    q0_ref[...] = xt[:, 0 * Q:1 * Q]
    q1_ref[...] = xt[:, 1 * Q:2 * Q]
    q2_ref[...] = xt[:, 2 * Q:3 * Q]
    q3_ref[...] = xt[:, 3 * Q:4 * Q]


_Q_SPEC = pl.BlockSpec((ROW_BLK, Q), lambda i: (i, 0))
_Q_SHAPE = jax.ShapeDtypeStruct((N, Q), jnp.float32)

_logmap = pl.pallas_call(
    _logmap_body,
    grid=(N // ROW_BLK,),
    in_specs=[pl.BlockSpec((ROW_BLK, D), lambda i: (i, 0))],
    out_specs=[_Q_SPEC, _Q_SPEC, _Q_SPEC, _Q_SPEC],
    out_shape=[_Q_SHAPE, _Q_SHAPE, _Q_SHAPE, _Q_SHAPE],
)


@functools.partial(
    pl.kernel,
    mesh=plsc.VectorSubcoreMesh(core_axis_name="c", subcore_axis_name="s"),
    compiler_params=pltpu.CompilerParams(use_tc_tiling_on_sc=False),
    out_type=[
        jax.ShapeDtypeStruct((NPAD, Q), jnp.float32),
        jax.ShapeDtypeStruct((NPAD, Q), jnp.float32),
        jax.ShapeDtypeStruct((NPAD, Q), jnp.float32),
        jax.ShapeDtypeStruct((NPAD, Q), jnp.float32),
        jax.ShapeDtypeStruct((NPAD, 16), jnp.float32),
        jax.ShapeDtypeStruct((NPAD, 16), jnp.float32),
    ],
    scratch_types=[
        pltpu.VMEM((NCHUNK, CHUNK), jnp.int32),
        pltpu.VMEM((NCHUNK, CHUNK), jnp.int32),
        pltpu.VMEM((CHUNK, Q), jnp.float32),
        pltpu.VMEM((CHUNK, Q), jnp.float32),
        pltpu.VMEM((CHUNK, Q), jnp.float32),
        pltpu.VMEM((CHUNK, Q), jnp.float32),
        pltpu.VMEM((CHUNK, Q), jnp.float32),
        pltpu.VMEM((CHUNK, Q), jnp.float32),
        pltpu.VMEM((CHUNK, Q), jnp.float32),
        pltpu.VMEM((CHUNK, Q), jnp.float32),
        pltpu.VMEM((CHUNK, 16), jnp.float32),
        pltpu.VMEM_SHARED((NPAD, Q), jnp.float32),
        pltpu.VMEM_SHARED((NPAD, 16), jnp.float32),
    ] + [pltpu.SemaphoreType.DMA] * 17,
)
def _sc_agg(xq0_hbm, xq1_hbm, xq2_hbm, xq3_hbm, src_hbm, dst_hbm,
            zrow_hbm, zdeg_hbm, ones_hbm,
            agg0_hbm, agg1_hbm, agg2_hbm, agg3_hbm, dega_hbm, degb_hbm,
            src_v, dst_v, a0, a1, a2, a3, b0, b1, b2, b3, ones_v,
            acc_sh, deg_sh,
            ga0, ga1, ga2, ga3, gb0, gb1, gb2, gb3,
            sa0, sa1, sa2, sa3, sb0, sb1, sb2, sb3, dsem):
    c = lax.axis_index("c")
    s = lax.axis_index("s")
    rows = pl.ds(s * RPW, RPW)
    bufs = [[a0, a1, a2, a3], [b0, b1, b2, b3]]
    gsems = [[ga0, ga1, ga2, ga3], [gb0, gb1, gb2, gb3]]
    ssems = [[sa0, sa1, sa2, sa3], [sb0, sb1, sb2, sb3]]

    pltpu.sync_copy(ones_hbm, ones_v)

    def one_pass(xq_hbm, agg_hbm, deg_half, deg_hbm):
        pltpu.sync_copy(src_hbm.at[s], src_v)
        pltpu.sync_copy(dst_hbm.at[s], dst_v)
        pltpu.sync_copy(zrow_hbm, acc_sh.at[rows])
        if deg_half is not None:
            pltpu.sync_copy(zdeg_hbm, deg_sh.at[rows])
        plsc.subcore_barrier()

        hdma = xq_hbm.at[pl.ds(0, CHUNK)]

        for h in range(2):
            for b in range(NBUF):
                pltpu.async_copy(
                    xq_hbm.at[src_v.at[h * NBUF + b]], bufs[h][b],
                    gsems[h][b])

        def scat_half(t, h, base):
            for b in range(NBUF):
                pltpu.make_async_copy(hdma, bufs[h][b], gsems[h][b]).wait()
                pltpu.async_copy(
                    bufs[h][b], acc_sh.at[dst_v.at[base + b]], ssems[h][b],
                    add=True)
            if deg_half is not None:
                @pl.when(deg_half(t))
                def _():
                    for b in range(NBUF):
                        pltpu.async_copy(
                            ones_v, deg_sh.at[dst_v.at[base + b]], dsem,
                            add=True)

        def refill_half(t, h, base):
            for b in range(NBUF):
                pltpu.make_async_copy(hdma, bufs[h][b], ssems[h][b]).wait()

                @pl.when(t < NROUND - 1)
                def _(h=h, b=b, base=base):
                    pltpu.async_copy(
                        xq_hbm.at[src_v.at[base + b]], bufs[h][b],
                        gsems[h][b])

        def rnd(t, carry):
            base = t * 2 * NBUF
            scat_half(t, 0, base)
            scat_half(t, 1, base + NBUF)
            refill_half(t, 0, base + 2 * NBUF)
            refill_half(t, 1, base + 3 * NBUF)
            if deg_half is not None:
                @pl.when(deg_half(t))
                def _():
                    for _b in range(2 * NBUF):
                        pltpu.make_async_copy(ones_hbm, ones_v, dsem).wait()
            return carry

        lax.fori_loop(0, NROUND, rnd, 0)
        plsc.subcore_barrier()
        pltpu.sync_copy(acc_sh.at[rows], agg_hbm.at[rows])
        if deg_half is not None:
            pltpu.sync_copy(deg_sh.at[rows], deg_hbm.at[rows])

    @pl.when(c == 0)
    def _():
        one_pass(xq0_hbm, agg0_hbm, lambda t: t < NROUND // 2, dega_hbm)
        one_pass(xq1_hbm, agg1_hbm, None, None)

    @pl.when(c == 1)
    def _():
        one_pass(xq2_hbm, agg2_hbm, lambda t: t >= NROUND // 2, degb_hbm)
        one_pass(xq3_hbm, agg3_hbm, None, None)


def _combine_body(q0_ref, q1_ref, q2_ref, q3_ref,
                  a0_ref, a1_ref, a2_ref, a3_ref, dega_ref, degb_ref,
                  ws_ref, wn_ref, b_ref, o_ref):
    deg = dega_ref[:, 0:1] + degb_ref[:, 0:1]
    inv = 1.0 / jnp.maximum(deg, 1.0)
    xt = jnp.concatenate(
        [q0_ref[...], q1_ref[...], q2_ref[...], q3_ref[...]], axis=1)
    ag = jnp.concatenate(
        [a0_ref[...], a1_ref[...], a2_ref[...], a3_ref[...]], axis=1)
    h = (jnp.dot(xt, ws_ref[...], preferred_element_type=jnp.float32)
         + jnp.dot(ag * inv, wn_ref[...], preferred_element_type=jnp.float32)
         + b_ref[...])
    nrm = jnp.sqrt(jnp.sum(h * h, axis=1, keepdims=True))
    nrm = jnp.maximum(nrm, EPS)
    o_ref[...] = jnp.tanh(nrm) * h / nrm


_W_SPEC = pl.BlockSpec((D, D), lambda i: (0, 0))
_DEG_SPEC = pl.BlockSpec((ROW_BLK, 16), lambda i: (i, 0))

_combine = pl.pallas_call(
    _combine_body,
    grid=(N // ROW_BLK,),
    in_specs=[
        _Q_SPEC, _Q_SPEC, _Q_SPEC, _Q_SPEC,
        _Q_SPEC, _Q_SPEC, _Q_SPEC, _Q_SPEC,
        _DEG_SPEC, _DEG_SPEC,
        _W_SPEC, _W_SPEC,
        pl.BlockSpec((1, D), lambda i: (0, 0)),
    ],
    out_specs=pl.BlockSpec((ROW_BLK, D), lambda i: (i, 0)),
    out_shape=jax.ShapeDtypeStruct((N, D), jnp.float32),
)


def kernel(x, edge_index, W_self, b_self, W_neigh, b_neigh):
    src = edge_index[0].astype(jnp.int32)
    dst = edge_index[1].astype(jnp.int32)
    pad = E_PAD - E
    src2 = jnp.concatenate([src, jnp.zeros((pad,), jnp.int32)]).reshape(
        NS, NCHUNK, CHUNK)
    dst2 = jnp.concatenate([dst, jnp.full((pad,), TRASH, jnp.int32)]).reshape(
        NS, NCHUNK, CHUNK)

    q0, q1, q2, q3 = _logmap(x)

    zrow = jnp.zeros((RPW, Q), jnp.float32)
    zdeg = jnp.zeros((RPW, 16), jnp.float32)
    ones = jnp.ones((CHUNK, 16), jnp.float32)
    a0, a1, a2, a3, dega, degb = _sc_agg(
        q0, q1, q2, q3, src2, dst2, zrow, zdeg, ones)

    bias = (b_self + b_neigh).reshape(1, D)
    return _combine(q0, q1, q2, q3, a0, a1, a2, a3, dega, degb,
                    W_self.T, W_neigh.T, bias)
